# Initial kernel scaffold; baseline (speedup 1.0000x reference)
import jax
import jax.numpy as jnp
from jax.experimental import pallas as pl


def kernel(x_1, x_2, edge_index_1, edge_index_2, batch_1, batch_2, g1_conv_W, g1_conv_b, g1_bn1_g, g1_bn1_b, g1_a1, g1_lin_W, g1_lin_b, g1_bn2_g, g1_bn2_b, g1_a2, g2_conv_W, g2_conv_b, g2_bn1_g, g2_bn1_b, g2_a1, g2_lin_W, g2_lin_b, g2_bn2_g, g2_bn2_b, g2_a2, cat_bn_g, cat_bn_b, cat_W, cat_b):
    raise NotImplementedError("write your pallas kernel here")


# speedup vs baseline: 27.0949x; 27.0949x over previous
import functools

import jax
import jax.numpy as jnp
from jax import lax
from jax.experimental import pallas as pl
from jax.experimental.pallas import tpu as pltpu
from jax.experimental.pallas import tpu_sc as plsc

N = 100000
E = 3200000
F = 16
H = 64
B = 128

NC = 2
NS = 16
NW = NC * NS

EPT = E // NW
K = 128
NFULL = EPT // K
TAIL = EPT - NFULL * K

ROWS_PER_TILE = 6256

_mesh = lambda: plsc.VectorSubcoreMesh(
    core_axis_name="c", subcore_axis_name="s", num_cores=NC, num_subcores=NS
)


def _axc():
  return lax.axis_index("c")


def _axs():
  return lax.axis_index("s")


def _edge_accum_body(gather, u_hbm, src_hbm, dst_hbm, zeros_hbm, out_hbm,
                     srcb, dstb, rowb, srct, dstt, rowt, sem, acc_sh):
  c = _axc()
  s = _axs()

  row0 = jnp.minimum(s * ROWS_PER_TILE, N - ROWS_PER_TILE)
  pltpu.sync_copy(zeros_hbm.at[pl.ds(row0, ROWS_PER_TILE)],
                  acc_sh.at[pl.ds(row0, ROWS_PER_TILE)])
  if not gather:
    def _ones(i, _):
      rowb[i, :] = jnp.full((16,), 1.0, jnp.float32)
      return 0
    lax.fori_loop(0, K, _ones, 0)

    def _onest(i, _):
      rowt[i, :] = jnp.full((16,), 1.0, jnp.float32)
      return 0
    lax.fori_loop(0, TAIL, _onest, 0)
  plsc.subcore_barrier()

  base = (c * NS + s) * EPT

  def _body(j, _):
    off = base + j * K
    pltpu.sync_copy(dst_hbm.at[pl.ds(off, K)], dstb)
    if gather:
      pltpu.sync_copy(src_hbm.at[pl.ds(off, K)], srcb)
      pltpu.async_copy(u_hbm.at[srcb], rowb, sem).wait()
    pltpu.sync_copy(rowb, acc_sh.at[dstb], add=True)
    return 0
  lax.fori_loop(0, NFULL, _body, 0)

  toff = base + NFULL * K
  pltpu.sync_copy(dst_hbm.at[pl.ds(toff, TAIL)], dstt)
  if gather:
    pltpu.sync_copy(src_hbm.at[pl.ds(toff, TAIL)], srct)
    pltpu.async_copy(u_hbm.at[srct], rowt, sem).wait()
  pltpu.sync_copy(rowt, acc_sh.at[dstt], add=True)

  plsc.subcore_barrier()
  pltpu.sync_copy(acc_sh.at[pl.ds(row0, ROWS_PER_TILE)],
                  out_hbm.at[c, pl.ds(row0, ROWS_PER_TILE)])


_EDGE_SCRATCH = lambda: [
    pltpu.VMEM((K,), jnp.int32),
    pltpu.VMEM((K,), jnp.int32),
    pltpu.VMEM((K, F), jnp.float32),
    pltpu.VMEM((TAIL,), jnp.int32),
    pltpu.VMEM((TAIL,), jnp.int32),
    pltpu.VMEM((TAIL, F), jnp.float32),
    pltpu.SemaphoreType.DMA,
    pltpu.VMEM_SHARED((N, F), jnp.float32),
]


def _make_deg_kernel():
  def body(dst_hbm, zeros_hbm, out_hbm, *scratch):
    _edge_accum_body(False, None, None, dst_hbm, zeros_hbm, out_hbm, *scratch)

  return pl.kernel(
      body,
      out_type=jax.ShapeDtypeStruct((NC, N, F), jnp.float32),
      mesh=_mesh(),
      scratch_types=_EDGE_SCRATCH(),
      compiler_params=pltpu.CompilerParams(use_tc_tiling_on_sc=False),
  )


def _make_agg_kernel():
  def body(u_hbm, src_hbm, dst_hbm, zeros_hbm, out_hbm, *scratch):
    _edge_accum_body(True, u_hbm, src_hbm, dst_hbm, zeros_hbm, out_hbm,
                     *scratch)

  return pl.kernel(
      body,
      out_type=jax.ShapeDtypeStruct((NC, N, F), jnp.float32),
      mesh=_mesh(),
      scratch_types=_EDGE_SCRATCH(),
      compiler_params=pltpu.CompilerParams(use_tc_tiling_on_sc=False),
  )


TR = 2000


def _scale_body(x1, d10, d11, x2, d20, d21, u1, v1, u2, v2):
  for x, d0, d1, u, v in ((x1, d10, d11, u1, v1), (x2, d20, d21, u2, v2)):
    deg = d0[...] + d1[...] + 1.0
    dinv = 1.0 / jnp.sqrt(deg)
    v[...] = dinv
    xr = x[...].astype(jnp.bfloat16).astype(jnp.float32)
    u[...] = xr * dinv


def _scale_call(x1, dp1, x2, dp2):
  spec = pl.BlockSpec((TR, F), lambda i: (i, 0))
  return pl.pallas_call(
      _scale_body,
      grid=(N // TR,),
      in_specs=[spec] * 6,
      out_specs=[spec] * 4,
      out_shape=[jax.ShapeDtypeStruct((N, F), jnp.float32)] * 4,
  )(x1, dp1[0], dp1[1], x2, dp2[0], dp2[1])


def _conv_body(s0, s1, u, dinv, w, bias, h, stats):
  i = pl.program_id(0)
  agg = (s0[...] + s1[...] + u[...]) * dinv[...]
  wr = w[...].astype(jnp.bfloat16).astype(jnp.float32)
  hv = jnp.dot(agg, wr, precision=lax.Precision.HIGHEST,
               preferred_element_type=jnp.float32) + bias[...]
  h[...] = hv

  @pl.when(i == 0)
  def _():
    stats[...] = jnp.zeros_like(stats)

  acc = jnp.concatenate(
      [jnp.sum(hv, axis=0, keepdims=True),
       jnp.sum(hv * hv, axis=0, keepdims=True),
       jnp.zeros((6, H), jnp.float32)], axis=0)
  stats[...] += acc


def _conv_call(sp, u, dinv, w, bias):
  spec16 = pl.BlockSpec((TR, F), lambda i: (i, 0))
  return pl.pallas_call(
      _conv_body,
      grid=(N // TR,),
      in_specs=[spec16, spec16, spec16, spec16,
                pl.BlockSpec((F, H), lambda i: (0, 0)),
                pl.BlockSpec((1, H), lambda i: (0, 0))],
      out_specs=[pl.BlockSpec((TR, H), lambda i: (i, 0)),
                 pl.BlockSpec((8, H), lambda i: (0, 0))],
      out_shape=[jax.ShapeDtypeStruct((N, H), jnp.float32),
                 jax.ShapeDtypeStruct((8, H), jnp.float32)],
  )(sp[0], sp[1], u, dinv, w, bias.reshape(1, H))


SEG_SZ = 3128
SEG_R = 184
SEG_NCH = -(-SEG_SZ // SEG_R)


def _segmax_body(h1, h2, b1, b2, out_hbm, hbuf, bsm, acc):
  c = _axc()
  s = _axs()
  w = c * NS + s
  base = jnp.minimum(w * SEG_SZ, N - SEG_SZ)

  def _init(i, _):
    acc[pl.ds(i * 16, 16)] = jnp.full((16,), -jnp.inf, jnp.float32)
    return 0
  lax.fori_loop(0, 2 * B * H // 16, _init, 0)

  for br, (h_hbm, b_hbm) in enumerate(((h1, b1), (h2, b2))):
    aoff = br * (B * H)

    def _chunk(j, _):
      off = base + jnp.minimum(j * SEG_R, SEG_SZ - SEG_R)
      pltpu.sync_copy(h_hbm.at[pl.ds(off, SEG_R)], hbuf)
      pltpu.sync_copy(b_hbm.at[pl.ds(off, SEG_R)], bsm.at[pl.ds(0, SEG_R)])

      def _row(r, _):
        b = bsm[pl.ds(r, 16)][0]
        for k in range(H // 16):
          a0 = aoff + b * H + k * 16
          hrow = hbuf[r, pl.ds(k * 16, 16)]
          acc[pl.ds(a0, 16)] = jnp.maximum(acc[pl.ds(a0, 16)], hrow)
        return 0
      lax.fori_loop(0, SEG_R, _row, 0)
      return 0
    lax.fori_loop(0, SEG_NCH, _chunk, 0)

    pltpu.sync_copy(acc.at[pl.ds(aoff, B * H)], out_hbm.at[br, w])


def _make_segmax_kernel():
  return pl.kernel(
      _segmax_body,
      out_type=jax.ShapeDtypeStruct((2, NW, B * H), jnp.float32),
      mesh=_mesh(),
      scratch_types=[
          pltpu.VMEM((SEG_R, H), jnp.float32),
          pltpu.VMEM((SEG_R + 16,), jnp.int32),
          pltpu.VMEM((2 * B * H,), jnp.float32),
      ],
      compiler_params=pltpu.CompilerParams(use_tc_tiling_on_sc=False),
  )


def _prelu(x, a):
  return jnp.where(x >= 0, x, a * x)


def _dot_default(a, b):
  ar = a.astype(jnp.bfloat16).astype(jnp.float32)
  br = b.astype(jnp.bfloat16).astype(jnp.float32)
  return jnp.dot(ar, br, precision=lax.Precision.HIGHEST,
                 preferred_element_type=jnp.float32)


def _tail_body(mx, st1, st2,
               bn1g1, bn1b1, a1_1, linw1, linb1, bn2g1, bn2b1, a2_1,
               bn1g2, bn1b2, a1_2, linw2, linb2, bn2g2, bn2b2, a2_2,
               catg, catb, catw, catbb, out):
  ps = []
  for br, (st, bn1g, bn1b, a1, linw, linb, bn2g, bn2b, a2) in enumerate((
      (st1, bn1g1, bn1b1, a1_1, linw1, linb1, bn2g1, bn2b1, a2_1),
      (st2, bn1g2, bn1b2, a1_2, linw2, linb2, bn2g2, bn2b2, a2_2))):
    pm = jnp.max(mx[br], axis=0)
    m = st[0:1, :] / N
    v = st[1:2, :] / N - m * m
    pmn = (pm - m) / jnp.sqrt(v + 1e-5) * bn1g[...] + bn1b[...]
    pmn = _prelu(pmn, a1[0, 0])
    p = _dot_default(pmn, linw[...]) + linb[...]
    m2 = jnp.mean(p, axis=0, keepdims=True)
    v2 = jnp.mean((p - m2) ** 2, axis=0, keepdims=True)
    p = (p - m2) / jnp.sqrt(v2 + 1e-5) * bn2g[...] + bn2b[...]
    p = _prelu(p, a2[0, 0])
    ps.append(p)

  p1, p2 = ps
  mc = (jnp.sum(p1, axis=0, keepdims=True)
        + jnp.sum(p2, axis=0, keepdims=True)) / (2 * B)
  vc = (jnp.sum((p1 - mc) ** 2, axis=0, keepdims=True)
        + jnp.sum((p2 - mc) ** 2, axis=0, keepdims=True)) / (2 * B)
  sc = jnp.sqrt(vc + 1e-5)
  p1n = (p1 - mc) / sc * catg[...] + catb[...]
  p2n = (p2 - mc) / sc * catg[...] + catb[...]
  xc = jnp.concatenate([p1n, p2n], axis=1)
  out[...] = _dot_default(xc, catw[...]) + catbb[...]


def _tail_call(mx, st1, st2, args1, args2, catg, catb, catw, catbb):
  full = lambda shape: pl.BlockSpec(shape, lambda: tuple(0 for _ in shape))
  row = full((1, H))
  scal = full((1, 1))
  ins = [mx.reshape(2, NW, B, H), st1, st2]
  specs = [full((2, NW, B, H)), full((8, H)), full((8, H))]
  for (bn1g, bn1b, a1, linw, linb, bn2g, bn2b, a2) in (args1, args2):
    ins += [bn1g.reshape(1, H), bn1b.reshape(1, H), a1.reshape(1, 1),
            linw, linb.reshape(1, H), bn2g.reshape(1, H),
            bn2b.reshape(1, H), a2.reshape(1, 1)]
    specs += [row, row, scal, full((H, H)), row, row, row, scal]
  ins += [catg.reshape(1, H), catb.reshape(1, H), catw, catbb.reshape(1, H)]
  specs += [row, row, full((2 * H, H)), row]
  return pl.pallas_call(
      _tail_body,
      in_specs=specs,
      out_specs=full((B, H)),
      out_shape=jax.ShapeDtypeStruct((B, H), jnp.float32),
  )(*ins)


def kernel(x_1, x_2, edge_index_1, edge_index_2, batch_1, batch_2,
           g1_conv_W, g1_conv_b, g1_bn1_g, g1_bn1_b, g1_a1, g1_lin_W,
           g1_lin_b, g1_bn2_g, g1_bn2_b, g1_a2,
           g2_conv_W, g2_conv_b, g2_bn1_g, g2_bn1_b, g2_a1, g2_lin_W,
           g2_lin_b, g2_bn2_g, g2_bn2_b, g2_a2,
           cat_bn_g, cat_bn_b, cat_W, cat_b):
  zeros = jnp.zeros((N, F), jnp.float32)
  src1, dst1 = edge_index_1[0], edge_index_1[1]
  src2, dst2 = edge_index_2[0], edge_index_2[1]

  deg_k = _make_deg_kernel()
  dp1 = deg_k(dst1, zeros)
  dp2 = deg_k(dst2, zeros)

  u1, v1, u2, v2 = _scale_call(x_1, dp1, x_2, dp2)

  agg_k = _make_agg_kernel()
  sp1 = agg_k(u1, src1, dst1, zeros)
  sp2 = agg_k(u2, src2, dst2, zeros)

  h1, st1 = _conv_call(sp1, u1, v1, g1_conv_W, g1_conv_b)
  h2, st2 = _conv_call(sp2, u2, v2, g2_conv_W, g2_conv_b)

  mx = _make_segmax_kernel()(h1, h2, batch_1, batch_2)

  return _tail_call(
      mx, st1, st2,
      (g1_bn1_g, g1_bn1_b, g1_a1, g1_lin_W, g1_lin_b, g1_bn2_g, g1_bn2_b,
       g1_a2),
      (g2_bn1_g, g2_bn1_b, g2_a1, g2_lin_W, g2_lin_b, g2_bn2_g, g2_bn2_b,
       g2_a2),
      cat_bn_g, cat_bn_b, cat_W, cat_b)


# speedup vs baseline: 69.3762x; 2.5605x over previous
import functools

import jax
import jax.numpy as jnp
from jax import lax
from jax.experimental import pallas as pl
from jax.experimental.pallas import tpu as pltpu
from jax.experimental.pallas import tpu_sc as plsc

N = 100000
E = 3200000
F = 16
H = 64
B = 128

NC = 2
NS = 16
NW = NC * NS

EPT = E // NW
K = 128
NFULL = EPT // K
TAIL = EPT - NFULL * K

ROWS_PER_TILE = 6256

_mesh = lambda: plsc.VectorSubcoreMesh(
    core_axis_name="c", subcore_axis_name="s", num_cores=NC, num_subcores=NS
)


def _axc():
  return lax.axis_index("c")


def _axs():
  return lax.axis_index("s")


def _edge_accum_body(gather, u_hbm, src_hbm, dst_hbm, zeros_hbm, out_hbm,
                     srcb, dstb, rowb, srct, dstt, rowt,
                     isem_s, isem_d, gsem, sem, acc_sh):
  c = _axc()
  s = _axs()

  row0 = jnp.minimum(s * ROWS_PER_TILE, N - ROWS_PER_TILE)
  pltpu.sync_copy(zeros_hbm.at[pl.ds(row0, ROWS_PER_TILE)],
                  acc_sh.at[pl.ds(row0, ROWS_PER_TILE)])
  if not gather:
    def _ones(i, _):
      rowb[0, i, :] = jnp.full((16,), 1.0, jnp.float32)
      return 0
    lax.fori_loop(0, K, _ones, 0)

    def _onest(i, _):
      rowt[i, :] = jnp.full((16,), 1.0, jnp.float32)
      return 0
    lax.fori_loop(0, TAIL, _onest, 0)
  plsc.subcore_barrier()

  base = (c * NS + s) * EPT

  def _idx_copies(j, r):
    off = base + j * K
    cp = [pltpu.make_async_copy(dst_hbm.at[pl.ds(off, K)], dstb.at[r],
                                isem_d.at[r])]
    if gather:
      cp.append(pltpu.make_async_copy(src_hbm.at[pl.ds(off, K)], srcb.at[r],
                                      isem_s.at[r]))
    return cp

  def _gather_copy(r):
    return pltpu.make_async_copy(u_hbm.at[srcb.at[r]], rowb.at[r],
                                 gsem.at[r])

  GLAG = 3
  SLAG = 5 if gather else 3
  DEPTH = RING

  def _step(j, _):
    @pl.when(j < NFULL)
    def _():
      for cp in _idx_copies(j, j % DEPTH):
        cp.start()

    if gather:
      g = j - GLAG
      @pl.when((j >= GLAG) & (g < NFULL))
      def _():
        r = g % DEPTH
        for cp in _idx_copies(g, r):
          cp.wait()
        _gather_copy(r).start()

    sc = j - SLAG
    @pl.when((j >= SLAG) & (sc < NFULL))
    def _():
      r = sc % DEPTH
      if gather:
        _gather_copy(r).wait()
        pltpu.sync_copy(rowb.at[r], acc_sh.at[dstb.at[r]], add=True)
      else:
        for cp in _idx_copies(sc, r):
          cp.wait()
        pltpu.sync_copy(rowb.at[0], acc_sh.at[dstb.at[r]], add=True)
    return 0
  lax.fori_loop(0, NFULL + SLAG, _step, 0)

  toff = base + NFULL * K
  pltpu.sync_copy(dst_hbm.at[pl.ds(toff, TAIL)], dstt)
  if gather:
    pltpu.sync_copy(src_hbm.at[pl.ds(toff, TAIL)], srct)
    pltpu.async_copy(u_hbm.at[srct], rowt, sem).wait()
  pltpu.sync_copy(rowt, acc_sh.at[dstt], add=True)

  plsc.subcore_barrier()
  pltpu.sync_copy(acc_sh.at[pl.ds(row0, ROWS_PER_TILE)],
                  out_hbm.at[c, pl.ds(row0, ROWS_PER_TILE)])


RING = 8

_EDGE_SCRATCH = lambda: [
    pltpu.VMEM((RING, K), jnp.int32),
    pltpu.VMEM((RING, K), jnp.int32),
    pltpu.VMEM((RING, K, F), jnp.float32),
    pltpu.VMEM((TAIL,), jnp.int32),
    pltpu.VMEM((TAIL,), jnp.int32),
    pltpu.VMEM((TAIL, F), jnp.float32),
    pltpu.SemaphoreType.DMA((RING,)),
    pltpu.SemaphoreType.DMA((RING,)),
    pltpu.SemaphoreType.DMA((RING,)),
    pltpu.SemaphoreType.DMA,
    pltpu.VMEM_SHARED((N, F), jnp.float32),
]


def _make_deg_kernel():
  def body(dst_hbm, zeros_hbm, out_hbm, *scratch):
    _edge_accum_body(False, None, None, dst_hbm, zeros_hbm, out_hbm, *scratch)

  return pl.kernel(
      body,
      out_type=jax.ShapeDtypeStruct((NC, N, F), jnp.float32),
      mesh=_mesh(),
      scratch_types=_EDGE_SCRATCH(),
      compiler_params=pltpu.CompilerParams(use_tc_tiling_on_sc=False),
  )


def _make_agg_kernel():
  def body(u_hbm, src_hbm, dst_hbm, zeros_hbm, out_hbm, *scratch):
    _edge_accum_body(True, u_hbm, src_hbm, dst_hbm, zeros_hbm, out_hbm,
                     *scratch)

  return pl.kernel(
      body,
      out_type=jax.ShapeDtypeStruct((NC, N, F), jnp.float32),
      mesh=_mesh(),
      scratch_types=_EDGE_SCRATCH(),
      compiler_params=pltpu.CompilerParams(use_tc_tiling_on_sc=False),
  )


TR = 2000


def _scale_body(x1, d10, d11, x2, d20, d21, u1, v1, u2, v2):
  for x, d0, d1, u, v in ((x1, d10, d11, u1, v1), (x2, d20, d21, u2, v2)):
    deg = d0[...] + d1[...] + 1.0
    dinv = 1.0 / jnp.sqrt(deg)
    v[...] = dinv
    xr = x[...].astype(jnp.bfloat16).astype(jnp.float32)
    u[...] = xr * dinv


def _scale_call(x1, dp1, x2, dp2):
  spec = pl.BlockSpec((TR, F), lambda i: (i, 0))
  return pl.pallas_call(
      _scale_body,
      grid=(N // TR,),
      in_specs=[spec] * 6,
      out_specs=[spec] * 4,
      out_shape=[jax.ShapeDtypeStruct((N, F), jnp.float32)] * 4,
  )(x1, dp1[0], dp1[1], x2, dp2[0], dp2[1])


def _conv_body(s0, s1, u, dinv, w, bias, h, stats):
  i = pl.program_id(0)
  agg = (s0[...] + s1[...] + u[...]) * dinv[...]
  wr = w[...].astype(jnp.bfloat16).astype(jnp.float32)
  hv = jnp.dot(agg, wr, precision=lax.Precision.HIGHEST,
               preferred_element_type=jnp.float32) + bias[...]
  h[...] = hv

  @pl.when(i == 0)
  def _():
    stats[...] = jnp.zeros_like(stats)

  acc = jnp.concatenate(
      [jnp.sum(hv, axis=0, keepdims=True),
       jnp.sum(hv * hv, axis=0, keepdims=True),
       jnp.zeros((6, H), jnp.float32)], axis=0)
  stats[...] += acc


def _conv_call(sp, u, dinv, w, bias):
  spec16 = pl.BlockSpec((TR, F), lambda i: (i, 0))
  return pl.pallas_call(
      _conv_body,
      grid=(N // TR,),
      in_specs=[spec16, spec16, spec16, spec16,
                pl.BlockSpec((F, H), lambda i: (0, 0)),
                pl.BlockSpec((1, H), lambda i: (0, 0))],
      out_specs=[pl.BlockSpec((TR, H), lambda i: (i, 0)),
                 pl.BlockSpec((8, H), lambda i: (0, 0))],
      out_shape=[jax.ShapeDtypeStruct((N, H), jnp.float32),
                 jax.ShapeDtypeStruct((8, H), jnp.float32)],
  )(sp[0], sp[1], u, dinv, w, bias.reshape(1, H))


SEG_SZ = 3128
SEG_R = 184
SEG_NCH = -(-SEG_SZ // SEG_R)


def _segmax_body(h1, h2, b1, b2, out_hbm, hbuf, bsm, acc):
  c = _axc()
  s = _axs()
  w = c * NS + s
  base = jnp.minimum(w * SEG_SZ, N - SEG_SZ)

  def _init(i, _):
    acc[pl.ds(i * 16, 16)] = jnp.full((16,), -jnp.inf, jnp.float32)
    return 0
  lax.fori_loop(0, 2 * B * H // 16, _init, 0)

  for br, (h_hbm, b_hbm) in enumerate(((h1, b1), (h2, b2))):
    aoff = br * (B * H)

    def _chunk(j, _):
      off = base + jnp.minimum(j * SEG_R, SEG_SZ - SEG_R)
      pltpu.sync_copy(h_hbm.at[pl.ds(off, SEG_R)], hbuf)
      pltpu.sync_copy(b_hbm.at[pl.ds(off, SEG_R)], bsm.at[pl.ds(0, SEG_R)])

      def _row(r, _):
        b = bsm[pl.ds(r, 16)][0]
        for k in range(H // 16):
          a0 = aoff + b * H + k * 16
          hrow = hbuf[r, pl.ds(k * 16, 16)]
          acc[pl.ds(a0, 16)] = jnp.maximum(acc[pl.ds(a0, 16)], hrow)
        return 0
      lax.fori_loop(0, SEG_R, _row, 0)
      return 0
    lax.fori_loop(0, SEG_NCH, _chunk, 0)

    pltpu.sync_copy(acc.at[pl.ds(aoff, B * H)], out_hbm.at[br, w])


def _make_segmax_kernel():
  return pl.kernel(
      _segmax_body,
      out_type=jax.ShapeDtypeStruct((2, NW, B * H), jnp.float32),
      mesh=_mesh(),
      scratch_types=[
          pltpu.VMEM((SEG_R, H), jnp.float32),
          pltpu.VMEM((SEG_R + 16,), jnp.int32),
          pltpu.VMEM((2 * B * H,), jnp.float32),
      ],
      compiler_params=pltpu.CompilerParams(use_tc_tiling_on_sc=False),
  )


def _prelu(x, a):
  return jnp.where(x >= 0, x, a * x)


def _dot_default(a, b):
  ar = a.astype(jnp.bfloat16).astype(jnp.float32)
  br = b.astype(jnp.bfloat16).astype(jnp.float32)
  return jnp.dot(ar, br, precision=lax.Precision.HIGHEST,
                 preferred_element_type=jnp.float32)


def _tail_body(mx, st1, st2,
               bn1g1, bn1b1, a1_1, linw1, linb1, bn2g1, bn2b1, a2_1,
               bn1g2, bn1b2, a1_2, linw2, linb2, bn2g2, bn2b2, a2_2,
               catg, catb, catw, catbb, out):
  ps = []
  for br, (st, bn1g, bn1b, a1, linw, linb, bn2g, bn2b, a2) in enumerate((
      (st1, bn1g1, bn1b1, a1_1, linw1, linb1, bn2g1, bn2b1, a2_1),
      (st2, bn1g2, bn1b2, a1_2, linw2, linb2, bn2g2, bn2b2, a2_2))):
    pm = jnp.max(mx[br], axis=0)
    m = st[0:1, :] / N
    v = st[1:2, :] / N - m * m
    pmn = (pm - m) / jnp.sqrt(v + 1e-5) * bn1g[...] + bn1b[...]
    pmn = _prelu(pmn, a1[0, 0])
    p = _dot_default(pmn, linw[...]) + linb[...]
    m2 = jnp.mean(p, axis=0, keepdims=True)
    v2 = jnp.mean((p - m2) ** 2, axis=0, keepdims=True)
    p = (p - m2) / jnp.sqrt(v2 + 1e-5) * bn2g[...] + bn2b[...]
    p = _prelu(p, a2[0, 0])
    ps.append(p)

  p1, p2 = ps
  mc = (jnp.sum(p1, axis=0, keepdims=True)
        + jnp.sum(p2, axis=0, keepdims=True)) / (2 * B)
  vc = (jnp.sum((p1 - mc) ** 2, axis=0, keepdims=True)
        + jnp.sum((p2 - mc) ** 2, axis=0, keepdims=True)) / (2 * B)
  sc = jnp.sqrt(vc + 1e-5)
  p1n = (p1 - mc) / sc * catg[...] + catb[...]
  p2n = (p2 - mc) / sc * catg[...] + catb[...]
  xc = jnp.concatenate([p1n, p2n], axis=1)
  out[...] = _dot_default(xc, catw[...]) + catbb[...]


def _tail_call(mx, st1, st2, args1, args2, catg, catb, catw, catbb):
  full = lambda shape: pl.BlockSpec(shape, lambda: tuple(0 for _ in shape))
  row = full((1, H))
  scal = full((1, 1))
  ins = [mx.reshape(2, NW, B, H), st1, st2]
  specs = [full((2, NW, B, H)), full((8, H)), full((8, H))]
  for (bn1g, bn1b, a1, linw, linb, bn2g, bn2b, a2) in (args1, args2):
    ins += [bn1g.reshape(1, H), bn1b.reshape(1, H), a1.reshape(1, 1),
            linw, linb.reshape(1, H), bn2g.reshape(1, H),
            bn2b.reshape(1, H), a2.reshape(1, 1)]
    specs += [row, row, scal, full((H, H)), row, row, row, scal]
  ins += [catg.reshape(1, H), catb.reshape(1, H), catw, catbb.reshape(1, H)]
  specs += [row, row, full((2 * H, H)), row]
  return pl.pallas_call(
      _tail_body,
      in_specs=specs,
      out_specs=full((B, H)),
      out_shape=jax.ShapeDtypeStruct((B, H), jnp.float32),
  )(*ins)


def kernel(x_1, x_2, edge_index_1, edge_index_2, batch_1, batch_2,
           g1_conv_W, g1_conv_b, g1_bn1_g, g1_bn1_b, g1_a1, g1_lin_W,
           g1_lin_b, g1_bn2_g, g1_bn2_b, g1_a2,
           g2_conv_W, g2_conv_b, g2_bn1_g, g2_bn1_b, g2_a1, g2_lin_W,
           g2_lin_b, g2_bn2_g, g2_bn2_b, g2_a2,
           cat_bn_g, cat_bn_b, cat_W, cat_b):
  zeros = jnp.zeros((N, F), jnp.float32)
  src1, dst1 = edge_index_1[0], edge_index_1[1]
  src2, dst2 = edge_index_2[0], edge_index_2[1]

  deg_k = _make_deg_kernel()
  dp1 = deg_k(dst1, zeros)
  dp2 = deg_k(dst2, zeros)

  u1, v1, u2, v2 = _scale_call(x_1, dp1, x_2, dp2)

  agg_k = _make_agg_kernel()
  sp1 = agg_k(u1, src1, dst1, zeros)
  sp2 = agg_k(u2, src2, dst2, zeros)

  h1, st1 = _conv_call(sp1, u1, v1, g1_conv_W, g1_conv_b)
  h2, st2 = _conv_call(sp2, u2, v2, g2_conv_W, g2_conv_b)

  mx = _make_segmax_kernel()(h1, h2, batch_1, batch_2)

  return _tail_call(
      mx, st1, st2,
      (g1_bn1_g, g1_bn1_b, g1_a1, g1_lin_W, g1_lin_b, g1_bn2_g, g1_bn2_b,
       g1_a2),
      (g2_bn1_g, g2_bn1_b, g2_a1, g2_lin_W, g2_lin_b, g2_bn2_g, g2_bn2_b,
       g2_a2),
      cat_bn_g, cat_bn_b, cat_W, cat_b)
